# Initial kernel scaffold; baseline (speedup 1.0000x reference)
import jax
import jax.numpy as jnp
from jax.experimental import pallas as pl


def kernel(attributes, E0, E1, E2, E3):
    raise NotImplementedError("write your pallas kernel here")


# speedup vs baseline: 2.4184x; 2.4184x over previous
import functools

import jax
import jax.numpy as jnp
from jax import lax
from jax.experimental import pallas as pl
from jax.experimental.pallas import tpu as pltpu
from jax.experimental.pallas import tpu_sc as plsc

B = 16384
BARS = 8
EDIM = 32
NTAB = 4
M = B * BARS * NTAB
NC = 2
NS = 16
NW = NC * NS
PER_W = M // NW
GCHUNK = 128
ROWS_W = PER_W // GCHUNK
WCHUNK = 1024
NG = WCHUNK // GCHUNK
NCH = PER_W // WCHUNK

BINS = 8

_mesh = plsc.VectorSubcoreMesh(core_axis_name="c", subcore_axis_name="s")


@functools.partial(
    pl.kernel,
    mesh=_mesh,
    compiler_params=pltpu.CompilerParams(use_tc_tiling_on_sc=False),
    out_type=jax.ShapeDtypeStruct((M, EDIM), jnp.float32),
    scratch_types=[
        pltpu.VMEM((ROWS_W, GCHUNK), jnp.int32),
        pltpu.VMEM((2, WCHUNK, EDIM), jnp.float32),
        pltpu.SemaphoreType.DMA,
        pltpu.SemaphoreType.DMA,
    ],
)
def _sc_gather(table_hbm, idx_hbm, out_hbm, idx_v, rows_v, gsem, osem):
    wid = lax.axis_index("s") * NC + lax.axis_index("c")
    base = wid * PER_W

    pltpu.sync_copy(idx_hbm.at[pl.ds(wid * ROWS_W, ROWS_W)], idx_v)

    offs = (lax.iota(jnp.int32, 16) % NTAB) * BINS

    def _add(t, carry):
        r = t // (GCHUNK // 16)
        i = t % (GCHUNK // 16)
        sl = pl.ds(i * 16, 16)
        idx_v[r, sl] = idx_v[r, sl] + offs
        return carry

    lax.fori_loop(0, ROWS_W * (GCHUNK // 16), _add, 0)

    def _chunk(c, carry):
        buf = c % 2
        @pl.when(c >= 2)
        def _():
            pltpu.make_async_copy(
                rows_v.at[buf],
                out_hbm.at[pl.ds(base + (c - 2) * WCHUNK, WCHUNK)],
                osem,
            ).wait()

        for g in range(NG):
            pltpu.async_copy(
                table_hbm.at[idx_v.at[c * NG + g]],
                rows_v.at[buf, pl.ds(g * GCHUNK, GCHUNK)],
                gsem,
            )
        for g in range(NG):
            pltpu.make_async_copy(
                table_hbm.at[idx_v.at[c * NG + g]],
                rows_v.at[buf, pl.ds(g * GCHUNK, GCHUNK)],
                gsem,
            ).wait()
        pltpu.async_copy(
            rows_v.at[buf],
            out_hbm.at[pl.ds(base + c * WCHUNK, WCHUNK)],
            osem,
        )
        return carry

    lax.fori_loop(0, NCH, _chunk, 0)

    for c in (NCH - 2, NCH - 1):
        pltpu.make_async_copy(
            rows_v.at[c % 2],
            out_hbm.at[pl.ds(base + c * WCHUNK, WCHUNK)],
            osem,
        ).wait()


def kernel(attributes, E0, E1, E2, E3):
    table = jnp.concatenate([E0, E1, E2, E3], axis=0)
    idx2d = attributes.reshape(M // GCHUNK, GCHUNK)
    out = _sc_gather(table, idx2d)
    return out.reshape(B, BARS, NTAB * EDIM)
